# baseline (speedup 1.0000x reference)
import functools
import math

import jax
import jax.numpy as jnp
from jax.experimental import pallas as pl
from jax.experimental.pallas import tpu as pltpu


_INV_SQRT2 = 1.0 / math.sqrt(2.0)


def _router_body(x_ref, sim_ref, gates_ref, w_ref):
    x = x_ref[...]
    s = sim_ref[...]
    xnorm = jnp.sqrt(jnp.sum(x * x, axis=1, keepdims=True))
    xn = x / jnp.maximum(xnorm, 1e-12)
    snorm = jnp.sqrt(jnp.sum(s * s, axis=0, keepdims=True))
    sn = s / jnp.maximum(snorm, 1e-12)
    logits = jnp.dot(xn, sn)
    thr = jax.nn.sigmoid(gates_ref[...])
    a = jnp.maximum(logits - thr, 0.0)
    active = a > 0.0
    amax = jnp.max(a, axis=1, keepdims=True)
    ex = jnp.where(active, jnp.exp(a - amax), 0.0)
    tot = jnp.sum(ex, axis=1, keepdims=True)
    w_ref[...] = ex / jnp.where(tot > 0.0, tot, 1.0)


def _ffn_body(counts_ref, idx_ref, x_ref, wgt_ref, w1_ref, w2_ref, out_ref,
              xg_ref, acc_ref, *, T_BLK, n_i):
    e = pl.program_id(0)
    i = pl.program_id(1)
    tb = pl.program_id(2)
    T = x_ref.shape[0]

    @pl.when((e == 0) & (i == 0) & (tb == 0))
    def _():
        out_ref[...] = jnp.zeros_like(out_ref)

    cnt = counts_ref[e]
    base = tb * T_BLK

    @pl.when(base < cnt)
    def _():
        @pl.when(i == 0)
        def _():
            def gather_row(r, carry):
                t = idx_ref[e * T + base + r]
                xg_ref[pl.ds(base + r, 1), :] = x_ref[pl.ds(t, 1), :]
                return carry
            jax.lax.fori_loop(0, T_BLK, gather_row, 0, unroll=8)

        xb = xg_ref[pl.ds(base, T_BLK), :].astype(jnp.bfloat16)
        w1b = w1_ref[0]
        h = jax.lax.dot_general(xb, w1b, (((1,), (1,)), ((), ())),
                                preferred_element_type=jnp.float32)
        h = 0.5 * h * (1.0 + jax.lax.erf(h * _INV_SQRT2))
        hw = (h * wgt_ref[0, pl.ds(base, T_BLK), :]).astype(jnp.bfloat16)
        w2b = w2_ref[0]
        contrib = jax.lax.dot_general(hw, w2b, (((1,), (1,)), ((), ())),
                                      preferred_element_type=jnp.float32)

        @pl.when(i == 0)
        def _():
            acc_ref[pl.ds(base, T_BLK), :] = contrib

        @pl.when(i > 0)
        def _():
            acc_ref[pl.ds(base, T_BLK), :] += contrib

        @pl.when(i == n_i - 1)
        def _():
            def scatter_row(r, carry):
                t = idx_ref[e * T + base + r]
                out_ref[pl.ds(t, 1), :] += acc_ref[pl.ds(base + r, 1), :]
                return carry
            jax.lax.fori_loop(0, T_BLK, scatter_row, 0, unroll=8)


def kernel(hidden_states, sim_matrix, gates, w1, w2):
    B, T, C = hidden_states.shape
    E, I, _ = w1.shape
    x = hidden_states.reshape(T, C)

    T_BLK = 256
    I_BLK = 512
    n_tb = T // T_BLK
    n_i = I // I_BLK

    wgt = pl.pallas_call(
        _router_body,
        grid=(n_tb,),
        in_specs=[
            pl.BlockSpec((T_BLK, C), lambda tb: (tb, 0)),
            pl.BlockSpec((C, E), lambda tb: (0, 0)),
            pl.BlockSpec((1, E), lambda tb: (0, 0)),
        ],
        out_specs=pl.BlockSpec((T_BLK, E), lambda tb: (tb, 0)),
        out_shape=jax.ShapeDtypeStruct((T, E), jnp.float32),
    )(x, sim_matrix, gates.reshape(1, E))

    active = wgt > 0.0
    counts = jnp.sum(active, axis=0).astype(jnp.int32)
    idx = jnp.argsort(~active, axis=0, stable=True).T.astype(jnp.int32)

    wgt_c = jnp.take_along_axis(wgt.T, idx, axis=1)

    out = pl.pallas_call(
        functools.partial(_ffn_body, T_BLK=T_BLK, n_i=n_i),
        grid_spec=pltpu.PrefetchScalarGridSpec(
            num_scalar_prefetch=2,
            grid=(E, n_i, n_tb),
            in_specs=[
                pl.BlockSpec((T, C), lambda e, i, tb, c_r, x_r: (0, 0)),
                pl.BlockSpec((1, T, 1), lambda e, i, tb, c_r, x_r: (e, 0, 0)),
                pl.BlockSpec((1, I_BLK, C), lambda e, i, tb, c_r, x_r: (e, i, 0)),
                pl.BlockSpec((1, C, I_BLK), lambda e, i, tb, c_r, x_r: (e, 0, i)),
            ],
            out_specs=pl.BlockSpec((T, C), lambda e, i, tb, c_r, x_r: (0, 0)),
            scratch_shapes=[
                pltpu.VMEM((T, C), jnp.float32),
                pltpu.VMEM((T, C), jnp.float32),
            ],
        ),
        out_shape=jax.ShapeDtypeStruct((T, C), jnp.float32),
        compiler_params=pltpu.CompilerParams(
            dimension_semantics=("arbitrary", "arbitrary", "arbitrary"),
        ),
    )(counts, idx.reshape(E * T), x,
      wgt_c.reshape(E, T, 1), w1.astype(jnp.bfloat16), w2.astype(jnp.bfloat16))

    return out.reshape(B, T, C)


# speedup vs baseline: 1.1132x; 1.1132x over previous
import functools
import math

import jax
import jax.numpy as jnp
from jax import lax
from jax.experimental import pallas as pl
from jax.experimental.pallas import tpu as pltpu
from jax.experimental.pallas import tpu_sc as plsc


_INV_SQRT2 = 1.0 / math.sqrt(2.0)
_LANES = 16


def _router_body(x_ref, sim_ref, gates_ref, w_ref):
    x = x_ref[...]
    s = sim_ref[...]
    xnorm = jnp.sqrt(jnp.sum(x * x, axis=1, keepdims=True))
    xn = x / jnp.maximum(xnorm, 1e-12)
    snorm = jnp.sqrt(jnp.sum(s * s, axis=0, keepdims=True))
    sn = s / jnp.maximum(snorm, 1e-12)
    logits = jnp.dot(xn, sn)
    thr = jax.nn.sigmoid(gates_ref[...])
    a = jnp.maximum(logits - thr, 0.0)
    active = a > 0.0
    amax = jnp.max(a, axis=1, keepdims=True)
    ex = jnp.where(active, jnp.exp(a - amax), 0.0)
    tot = jnp.sum(ex, axis=1, keepdims=True)
    w_ref[...] = (ex / jnp.where(tot > 0.0, tot, 1.0)).T


def _compact_body(wgt_hbm, idx_hbm, wgtc_hbm, counts_hbm,
                  col_v, idx_v, wgtc_v, cnt_v, *, E, T, NC):
    wid = lax.axis_index("s") * NC + lax.axis_index("c")

    @pl.when(wid < E)
    def _():
        pltpu.sync_copy(wgt_hbm.at[wid], col_v)
        one = jnp.ones((_LANES,), jnp.int32)
        zero = jnp.zeros((_LANES,), jnp.int32)

        def chunk(j, off_vec):
            base = j * _LANES
            idx_v[pl.ds(base, _LANES)] = zero
            wgtc_v[pl.ds(base, _LANES)] = jnp.zeros((_LANES,), jnp.float32)
            v = col_v[pl.ds(base, _LANES)]
            m = v > 0.0
            mi = jnp.where(m, one, zero)
            ids = lax.iota(jnp.int32, _LANES) + base
            pos = off_vec + plsc.cumsum(mi) - 1
            plsc.store_scatter(idx_v, [pos], ids, mask=m)
            plsc.store_scatter(wgtc_v, [pos], v, mask=m)
            return off_vec + plsc.all_reduce_population_count(m)

        cnt_v[...] = lax.fori_loop(0, T // _LANES, chunk, zero)
        pltpu.sync_copy(cnt_v, counts_hbm.at[wid])
        pltpu.sync_copy(idx_v, idx_hbm.at[wid])
        pltpu.sync_copy(wgtc_v, wgtc_hbm.at[wid])


def _ffn_body(counts_ref, idx_ref, x_ref, wgt_ref, w1_ref, w2_ref, out_ref,
              xg_ref, acc_ref, *, T_BLK, n_i):
    e = pl.program_id(0)
    i = pl.program_id(1)
    tb = pl.program_id(2)
    T = x_ref.shape[0]

    @pl.when((e == 0) & (i == 0) & (tb == 0))
    def _():
        out_ref[...] = jnp.zeros_like(out_ref)

    cnt = counts_ref[e, 0]
    base = tb * T_BLK

    @pl.when(base < cnt)
    def _():
        @pl.when(i == 0)
        def _():
            def gather_row(r, carry):
                t = idx_ref[e * T + base + r]
                xg_ref[pl.ds(base + r, 1), :] = x_ref[pl.ds(t, 1), :]
                return carry
            jax.lax.fori_loop(0, T_BLK, gather_row, 0, unroll=8)

        xb = xg_ref[pl.ds(base, T_BLK), :]
        w1b = w1_ref[0]
        h = jax.lax.dot_general(xb, w1b, (((1,), (1,)), ((), ())))
        h = 0.5 * h * (1.0 + jax.lax.erf(h * _INV_SQRT2))
        hw = h * wgt_ref[0, pl.ds(base, T_BLK), :]
        w2b = w2_ref[0]
        contrib = jax.lax.dot_general(hw, w2b, (((1,), (1,)), ((), ())))

        @pl.when(i == 0)
        def _():
            acc_ref[pl.ds(base, T_BLK), :] = contrib

        @pl.when(i > 0)
        def _():
            acc_ref[pl.ds(base, T_BLK), :] += contrib

        @pl.when(i == n_i - 1)
        def _():
            def scatter_row(r, carry):
                t = idx_ref[e * T + base + r]
                out_ref[pl.ds(t, 1), :] += acc_ref[pl.ds(base + r, 1), :]
                return carry
            jax.lax.fori_loop(0, T_BLK, scatter_row, 0, unroll=8)


def _router_call(x, sim_matrix, gates):
    T, C = x.shape
    E = sim_matrix.shape[1]
    T_BLK = 256
    n_tb = T // T_BLK
    return pl.pallas_call(
        _router_body,
        grid=(n_tb,),
        in_specs=[
            pl.BlockSpec((T_BLK, C), lambda tb: (tb, 0)),
            pl.BlockSpec((C, E), lambda tb: (0, 0)),
            pl.BlockSpec((1, E), lambda tb: (0, 0)),
        ],
        out_specs=pl.BlockSpec((E, T_BLK), lambda tb: (0, tb)),
        out_shape=jax.ShapeDtypeStruct((E, T), jnp.float32),
    )(x, sim_matrix, gates.reshape(1, E))


def _compact_call(wgt):
    E, T = wgt.shape
    active = wgt > 0.0
    counts = jnp.sum(active, axis=1).astype(jnp.int32)
    idx = jnp.argsort(~active, axis=1, stable=True).astype(jnp.int32)
    wgt_c = jnp.take_along_axis(wgt, idx, axis=1)
    counts16 = jnp.broadcast_to(counts[:, None], (E, _LANES))
    return idx, wgt_c, counts16


def _ffn_call(counts, idx, x, wgt_c, w1, w2):
    T, C = x.shape
    E, I, _ = w1.shape
    T_BLK = 256
    I_BLK = 512
    n_tb = T // T_BLK
    n_i = I // I_BLK
    return pl.pallas_call(
        functools.partial(_ffn_body, T_BLK=T_BLK, n_i=n_i),
        grid_spec=pltpu.PrefetchScalarGridSpec(
            num_scalar_prefetch=2,
            grid=(E, n_i, n_tb),
            in_specs=[
                pl.BlockSpec((T, C), lambda e, i, tb, c_r, x_r: (0, 0)),
                pl.BlockSpec((1, T, 1), lambda e, i, tb, c_r, x_r: (e, 0, 0)),
                pl.BlockSpec((1, I_BLK, C), lambda e, i, tb, c_r, x_r: (e, i, 0)),
                pl.BlockSpec((1, C, I_BLK), lambda e, i, tb, c_r, x_r: (e, 0, i)),
            ],
            out_specs=pl.BlockSpec((T, C), lambda e, i, tb, c_r, x_r: (0, 0)),
            scratch_shapes=[
                pltpu.VMEM((T, C), jnp.float32),
                pltpu.VMEM((T, C), jnp.float32),
            ],
        ),
        out_shape=jax.ShapeDtypeStruct((T, C), jnp.float32),
        compiler_params=pltpu.CompilerParams(
            dimension_semantics=("arbitrary", "arbitrary", "arbitrary"),
        ),
    )(counts, idx.reshape(E * T), x, wgt_c.reshape(E, T, 1), w1, w2)


def kernel(hidden_states, sim_matrix, gates, w1, w2):
    B, T, C = hidden_states.shape
    x = hidden_states.reshape(T, C)
    wgt = _router_call(x, sim_matrix, gates)
    idx, wgt_c, counts = _compact_call(wgt)
    out = _ffn_call(counts, idx, x, wgt_c, w1, w2)
    return out.reshape(B, T, C)


# speedup vs baseline: 1.1284x; 1.0136x over previous
import functools
import math

import jax
import jax.numpy as jnp
from jax import lax
from jax.experimental import pallas as pl
from jax.experimental.pallas import tpu as pltpu
from jax.experimental.pallas import tpu_sc as plsc


_INV_SQRT2 = 1.0 / math.sqrt(2.0)
_LANES = 16


def _router_body(x_ref, sim_ref, gates_ref, w_ref):
    x = x_ref[...]
    s = sim_ref[...]
    xnorm = jnp.sqrt(jnp.sum(x * x, axis=1, keepdims=True))
    xn = x / jnp.maximum(xnorm, 1e-12)
    snorm = jnp.sqrt(jnp.sum(s * s, axis=0, keepdims=True))
    sn = s / jnp.maximum(snorm, 1e-12)
    logits = jnp.dot(xn, sn)
    thr = jax.nn.sigmoid(gates_ref[...])
    a = jnp.maximum(logits - thr, 0.0)
    active = a > 0.0
    amax = jnp.max(a, axis=1, keepdims=True)
    ex = jnp.where(active, jnp.exp(a - amax), 0.0)
    tot = jnp.sum(ex, axis=1, keepdims=True)
    w_ref[...] = (ex / jnp.where(tot > 0.0, tot, 1.0)).T


def _compact_body(wgt_hbm, idx_hbm, wgtc_hbm, counts_hbm,
                  col_v, idx_v, wgtc_v, cnt_v, *, E, T, NC):
    wid = lax.axis_index("s") * NC + lax.axis_index("c")

    @pl.when(wid < E)
    def _():
        pltpu.sync_copy(wgt_hbm.at[wid], col_v)
        one = jnp.ones((_LANES,), jnp.int32)
        zero = jnp.zeros((_LANES,), jnp.int32)

        def chunk(j, off_vec):
            base = j * _LANES
            idx_v[pl.ds(base, _LANES)] = zero
            wgtc_v[pl.ds(base, _LANES)] = jnp.zeros((_LANES,), jnp.float32)
            v = col_v[pl.ds(base, _LANES)]
            m = v > 0.0
            mi = jnp.where(m, one, zero)
            ids = lax.iota(jnp.int32, _LANES) + base
            pos = off_vec + plsc.cumsum(mi) - 1
            plsc.store_scatter(idx_v, [pos], ids, mask=m)
            plsc.store_scatter(wgtc_v, [pos], v, mask=m)
            return off_vec + plsc.all_reduce_population_count(m)

        cnt_v[...] = lax.fori_loop(0, T // _LANES, chunk, zero)
        pltpu.sync_copy(cnt_v, counts_hbm.at[wid])
        pltpu.sync_copy(idx_v, idx_hbm.at[wid])
        pltpu.sync_copy(wgtc_v, wgtc_hbm.at[wid])


def _ffn_body(counts_ref, idx_ref, x_ref, wgt_ref, w1_ref, w2_ref, out_ref,
              xg_ref, acc_ref, *, T_BLK, n_i):
    e = pl.program_id(0)
    i = pl.program_id(1)
    tb = pl.program_id(2)
    T = x_ref.shape[0]

    @pl.when((e == 0) & (i == 0) & (tb == 0))
    def _():
        out_ref[...] = jnp.zeros_like(out_ref)
        xg_ref[...] = jnp.zeros_like(xg_ref)

    cnt = counts_ref[e, 0]
    base = tb * T_BLK

    @pl.when(base < cnt)
    def _():
        nrows = jnp.minimum(cnt - base, T_BLK)
        ngrp = (nrows + 15) // 16

        @pl.when(i == 0)
        def _():
            def gather_grp(o, carry):
                rbase = base + o * 16
                for k in range(16):
                    t = idx_ref[e * T + rbase + k]
                    xg_ref[pl.ds(rbase + k, 1), :] = x_ref[pl.ds(t, 1), :]
                return carry
            jax.lax.fori_loop(0, ngrp, gather_grp, 0)

        xb = xg_ref[pl.ds(base, T_BLK), :]
        w1b = w1_ref[0]
        h = jax.lax.dot_general(xb, w1b, (((1,), (1,)), ((), ())))
        h = 0.5 * h * (1.0 + jax.lax.erf(h * _INV_SQRT2))
        hw = h * wgt_ref[0, pl.ds(base, T_BLK), :]
        w2b = w2_ref[0]
        contrib = jax.lax.dot_general(hw, w2b, (((1,), (1,)), ((), ())))

        @pl.when(i == 0)
        def _():
            acc_ref[pl.ds(base, T_BLK), :] = contrib

        @pl.when(i > 0)
        def _():
            acc_ref[pl.ds(base, T_BLK), :] += contrib

        @pl.when(i == n_i - 1)
        def _():
            def scatter_grp(o, carry):
                rbase = base + o * 16
                for k in range(16):
                    t = idx_ref[e * T + rbase + k]
                    out_ref[pl.ds(t, 1), :] += acc_ref[pl.ds(rbase + k, 1), :]
                return carry
            jax.lax.fori_loop(0, ngrp, scatter_grp, 0)


def _router_call(x, sim_matrix, gates):
    T, C = x.shape
    E = sim_matrix.shape[1]
    T_BLK = 256
    n_tb = T // T_BLK
    return pl.pallas_call(
        _router_body,
        grid=(n_tb,),
        in_specs=[
            pl.BlockSpec((T_BLK, C), lambda tb: (tb, 0)),
            pl.BlockSpec((C, E), lambda tb: (0, 0)),
            pl.BlockSpec((1, E), lambda tb: (0, 0)),
        ],
        out_specs=pl.BlockSpec((E, T_BLK), lambda tb: (0, tb)),
        out_shape=jax.ShapeDtypeStruct((E, T), jnp.float32),
    )(x, sim_matrix, gates.reshape(1, E))


def _compact_call(wgt):
    E, T = wgt.shape
    active = wgt > 0.0
    counts = jnp.sum(active, axis=1).astype(jnp.int32)
    idx = jnp.argsort(~active, axis=1, stable=True).astype(jnp.int32)
    wgt_c = jnp.take_along_axis(wgt, idx, axis=1)
    counts16 = jnp.broadcast_to(counts[:, None], (E, _LANES))
    return idx, wgt_c, counts16


def _ffn_call(counts, idx, x, wgt_c, w1, w2):
    T, C = x.shape
    E, I, _ = w1.shape
    T_BLK = 256
    I_BLK = 512
    n_tb = T // T_BLK
    n_i = I // I_BLK
    return pl.pallas_call(
        functools.partial(_ffn_body, T_BLK=T_BLK, n_i=n_i),
        grid_spec=pltpu.PrefetchScalarGridSpec(
            num_scalar_prefetch=2,
            grid=(E, n_i, n_tb),
            in_specs=[
                pl.BlockSpec((T, C), lambda e, i, tb, c_r, x_r: (0, 0)),
                pl.BlockSpec((1, T, 1), lambda e, i, tb, c_r, x_r: (e, 0, 0)),
                pl.BlockSpec((1, I_BLK, C), lambda e, i, tb, c_r, x_r: (e, i, 0)),
                pl.BlockSpec((1, C, I_BLK), lambda e, i, tb, c_r, x_r: (e, 0, i)),
            ],
            out_specs=pl.BlockSpec((T, C), lambda e, i, tb, c_r, x_r: (0, 0)),
            scratch_shapes=[
                pltpu.VMEM((T, C), jnp.float32),
                pltpu.VMEM((T, C), jnp.float32),
            ],
        ),
        out_shape=jax.ShapeDtypeStruct((T, C), jnp.float32),
        compiler_params=pltpu.CompilerParams(
            dimension_semantics=("arbitrary", "arbitrary", "arbitrary"),
        ),
    )(counts, idx.reshape(E * T), x, wgt_c.reshape(E, T, 1), w1, w2)


def kernel(hidden_states, sim_matrix, gates, w1, w2):
    B, T, C = hidden_states.shape
    x = hidden_states.reshape(T, C)
    wgt = _router_call(x, sim_matrix, gates)
    idx, wgt_c, counts = _compact_call(wgt)
    out = _ffn_call(counts, idx, x, wgt_c, w1, w2)
    return out.reshape(B, T, C)


# speedup vs baseline: 1.2969x; 1.1494x over previous
import functools
import math

import jax
import jax.numpy as jnp
from jax import lax
from jax.experimental import pallas as pl
from jax.experimental.pallas import tpu as pltpu
from jax.experimental.pallas import tpu_sc as plsc


_INV_SQRT2 = 1.0 / math.sqrt(2.0)
_LANES = 16


def _router_body(x_ref, sim_ref, gates_ref, w_ref):
    x = x_ref[...]
    s = sim_ref[...]
    xnorm = jnp.sqrt(jnp.sum(x * x, axis=1, keepdims=True))
    xn = x / jnp.maximum(xnorm, 1e-12)
    snorm = jnp.sqrt(jnp.sum(s * s, axis=0, keepdims=True))
    sn = s / jnp.maximum(snorm, 1e-12)
    logits = jnp.dot(xn, sn)
    thr = jax.nn.sigmoid(gates_ref[...])
    a = jnp.maximum(logits - thr, 0.0)
    active = a > 0.0
    amax = jnp.max(a, axis=1, keepdims=True)
    ex = jnp.where(active, jnp.exp(a - amax), 0.0)
    tot = jnp.sum(ex, axis=1, keepdims=True)
    w_ref[...] = (ex / jnp.where(tot > 0.0, tot, 1.0)).T


def _compact_body(wgt_hbm, idx_hbm, wgtc_hbm, counts_hbm,
                  col_v, idx_v, wgtc_v, cnt_v, *, E, T, NC):
    wid = lax.axis_index("s") * NC + lax.axis_index("c")

    @pl.when(wid < E)
    def _():
        pltpu.sync_copy(wgt_hbm.at[wid], col_v)
        one = jnp.ones((_LANES,), jnp.int32)
        zero = jnp.zeros((_LANES,), jnp.int32)

        def chunk(j, off_vec):
            base = j * _LANES
            idx_v[pl.ds(base, _LANES)] = zero
            wgtc_v[pl.ds(base, _LANES)] = jnp.zeros((_LANES,), jnp.float32)
            v = col_v[pl.ds(base, _LANES)]
            m = v > 0.0
            mi = jnp.where(m, one, zero)
            ids = lax.iota(jnp.int32, _LANES) + base
            pos = off_vec + plsc.cumsum(mi) - 1
            plsc.store_scatter(idx_v, [pos], ids, mask=m)
            plsc.store_scatter(wgtc_v, [pos], v, mask=m)
            return off_vec + plsc.all_reduce_population_count(m)

        cnt_v[...] = lax.fori_loop(0, T // _LANES, chunk, zero)
        pltpu.sync_copy(cnt_v, counts_hbm.at[wid])
        pltpu.sync_copy(idx_v, idx_hbm.at[wid])
        pltpu.sync_copy(wgtc_v, wgtc_hbm.at[wid])


def _ffn_body(counts_ref, idx_ref, x_ref, wgt_ref, w1_ref, w2_ref, out_ref,
              xg_ref, acc_ref, *, T_BLK, n_i):
    e = pl.program_id(0)
    i = pl.program_id(1)
    tb = pl.program_id(2)
    T = x_ref.shape[0]

    @pl.when((e == 0) & (i == 0) & (tb == 0))
    def _():
        out_ref[...] = jnp.zeros_like(out_ref)
        xg_ref[...] = jnp.zeros_like(xg_ref)

    cnt = counts_ref[e, 0]
    base = tb * T_BLK

    @pl.when(base < cnt)
    def _():
        nrows = jnp.minimum(cnt - base, T_BLK)
        ngrp = (nrows + 15) // 16

        @pl.when(i == 0)
        def _():
            def gather_grp(o, carry):
                rbase = base + o * 16
                for k in range(16):
                    t = idx_ref[e * T + rbase + k]
                    xg_ref[pl.ds(rbase + k, 1), :] = x_ref[pl.ds(t, 1), :]
                return carry
            jax.lax.fori_loop(0, ngrp, gather_grp, 0)

        xb = xg_ref[pl.ds(base, T_BLK), :]
        w1b = w1_ref[0]
        h = jax.lax.dot_general(xb, w1b, (((1,), (1,)), ((), ())))
        h = 0.5 * h * (1.0 + jax.lax.erf(h * _INV_SQRT2))
        hw = h * wgt_ref[0, pl.ds(base, T_BLK), :]
        w2b = w2_ref[0]
        contrib = jax.lax.dot_general(hw, w2b, (((1,), (1,)), ((), ())))

        @pl.when(i == 0)
        def _():
            acc_ref[pl.ds(base, T_BLK), :] = contrib

        @pl.when(i > 0)
        def _():
            acc_ref[pl.ds(base, T_BLK), :] += contrib

        @pl.when(i == n_i - 1)
        def _():
            def scatter_grp(o, carry):
                rbase = base + o * 16
                for k in range(16):
                    t = idx_ref[e * T + rbase + k]
                    out_ref[pl.ds(t, 1), :] += acc_ref[pl.ds(rbase + k, 1), :]
                return carry
            jax.lax.fori_loop(0, ngrp, scatter_grp, 0)


def _router_call(x, sim_matrix, gates):
    T, C = x.shape
    E = sim_matrix.shape[1]
    T_BLK = 256
    n_tb = T // T_BLK
    return pl.pallas_call(
        _router_body,
        grid=(n_tb,),
        in_specs=[
            pl.BlockSpec((T_BLK, C), lambda tb: (tb, 0)),
            pl.BlockSpec((C, E), lambda tb: (0, 0)),
            pl.BlockSpec((1, E), lambda tb: (0, 0)),
        ],
        out_specs=pl.BlockSpec((E, T_BLK), lambda tb: (0, tb)),
        out_shape=jax.ShapeDtypeStruct((E, T), jnp.float32),
    )(x, sim_matrix, gates.reshape(1, E))


def _compact_call(wgt):
    E, T = wgt.shape
    active = wgt > 0.0
    counts = jnp.sum(active, axis=1).astype(jnp.int32)
    idx = jnp.argsort(~active, axis=1, stable=True).astype(jnp.int32)
    wgt_c = jnp.take_along_axis(wgt, idx, axis=1)
    counts16 = jnp.broadcast_to(counts[:, None], (E, _LANES))
    return idx, wgt_c, counts16


def _ffn_call(counts, idx, x, wgt_c, w1, w2):
    T, C = x.shape
    E, I, _ = w1.shape
    T_BLK = 512
    I_BLK = 512
    n_tb = T // T_BLK
    n_i = I // I_BLK
    return pl.pallas_call(
        functools.partial(_ffn_body, T_BLK=T_BLK, n_i=n_i),
        grid_spec=pltpu.PrefetchScalarGridSpec(
            num_scalar_prefetch=2,
            grid=(E, n_i, n_tb),
            in_specs=[
                pl.BlockSpec((T, C), lambda e, i, tb, c_r, x_r: (0, 0)),
                pl.BlockSpec((1, T, 1), lambda e, i, tb, c_r, x_r: (e, 0, 0)),
                pl.BlockSpec((1, I_BLK, C), lambda e, i, tb, c_r, x_r: (e, i, 0)),
                pl.BlockSpec((1, C, I_BLK), lambda e, i, tb, c_r, x_r: (e, 0, i)),
            ],
            out_specs=pl.BlockSpec((T, C), lambda e, i, tb, c_r, x_r: (0, 0)),
            scratch_shapes=[
                pltpu.VMEM((T, C), jnp.float32),
                pltpu.VMEM((T, C), jnp.float32),
            ],
        ),
        out_shape=jax.ShapeDtypeStruct((T, C), jnp.float32),
        compiler_params=pltpu.CompilerParams(
            dimension_semantics=("arbitrary", "arbitrary", "arbitrary"),
        ),
    )(counts, idx.reshape(E * T), x, wgt_c.reshape(E, T, 1), w1, w2)


def kernel(hidden_states, sim_matrix, gates, w1, w2):
    B, T, C = hidden_states.shape
    x = hidden_states.reshape(T, C)
    wgt = _router_call(x, sim_matrix, gates)
    idx, wgt_c, counts = _compact_call(wgt)
    out = _ffn_call(counts, idx, x, wgt_c, w1, w2)
    return out.reshape(B, T, C)


# speedup vs baseline: 1.4715x; 1.1346x over previous
import functools
import math

import jax
import jax.numpy as jnp
from jax import lax
from jax.experimental import pallas as pl
from jax.experimental.pallas import tpu as pltpu
from jax.experimental.pallas import tpu_sc as plsc


_INV_SQRT2 = 1.0 / math.sqrt(2.0)
_LANES = 16


def _router_body(x_ref, sim_ref, gates_ref, w_ref):
    x = x_ref[...]
    s = sim_ref[...]
    xnorm = jnp.sqrt(jnp.sum(x * x, axis=1, keepdims=True))
    xn = x / jnp.maximum(xnorm, 1e-12)
    snorm = jnp.sqrt(jnp.sum(s * s, axis=0, keepdims=True))
    sn = s / jnp.maximum(snorm, 1e-12)
    logits = jnp.dot(xn, sn)
    thr = jax.nn.sigmoid(gates_ref[...])
    a = jnp.maximum(logits - thr, 0.0)
    active = a > 0.0
    amax = jnp.max(a, axis=1, keepdims=True)
    ex = jnp.where(active, jnp.exp(a - amax), 0.0)
    tot = jnp.sum(ex, axis=1, keepdims=True)
    w_ref[...] = (ex / jnp.where(tot > 0.0, tot, 1.0)).T


def _compact_body(wgt_hbm, idx_hbm, wgtc_hbm, counts_hbm,
                  col_v, idx_v, wgtc_v, cnt_v, *, E, T, NC):
    wid = lax.axis_index("s") * NC + lax.axis_index("c")

    @pl.when(wid < E)
    def _():
        pltpu.sync_copy(wgt_hbm.at[wid], col_v)
        one = jnp.ones((_LANES,), jnp.int32)
        zero = jnp.zeros((_LANES,), jnp.int32)

        def chunk(j, off_vec):
            base = j * _LANES
            idx_v[pl.ds(base, _LANES)] = zero
            wgtc_v[pl.ds(base, _LANES)] = jnp.zeros((_LANES,), jnp.float32)
            v = col_v[pl.ds(base, _LANES)]
            m = v > 0.0
            mi = jnp.where(m, one, zero)
            ids = lax.iota(jnp.int32, _LANES) + base
            pos = off_vec + plsc.cumsum(mi) - 1
            plsc.store_scatter(idx_v, [pos], ids, mask=m)
            plsc.store_scatter(wgtc_v, [pos], v, mask=m)
            return off_vec + plsc.all_reduce_population_count(m)

        cnt_v[...] = lax.fori_loop(0, T // _LANES, chunk, zero)
        pltpu.sync_copy(cnt_v, counts_hbm.at[wid])
        pltpu.sync_copy(idx_v, idx_hbm.at[wid])
        pltpu.sync_copy(wgtc_v, wgtc_hbm.at[wid])


def _ffn_body(counts_ref, idx_ref, x_ref, wgt_ref, w1_ref, w2_ref, out_ref,
              xg_ref, acc_ref, *, T_BLK, n_i):
    e = pl.program_id(0)
    i = pl.program_id(1)
    tb = pl.program_id(2)
    T = x_ref.shape[0]

    @pl.when((e == 0) & (i == 0) & (tb == 0))
    def _():
        out_ref[...] = jnp.zeros_like(out_ref)
        xg_ref[...] = jnp.zeros_like(xg_ref)

    cnt = counts_ref[e, 0]
    base = tb * T_BLK

    @pl.when(base < cnt)
    def _():
        nrows = jnp.minimum(cnt - base, T_BLK)
        ngrp = (nrows + 15) // 16

        @pl.when(i == 0)
        def _():
            def gather_grp(o, carry):
                rbase = base + o * 16
                for k in range(16):
                    t = idx_ref[e * T + rbase + k]
                    xg_ref[pl.ds(rbase + k, 1), :] = x_ref[pl.ds(t, 1), :]
                return carry
            jax.lax.fori_loop(0, ngrp, gather_grp, 0)

        xb = xg_ref[pl.ds(base, T_BLK), :]
        w1b = w1_ref[0]
        h = jax.lax.dot_general(xb, w1b, (((1,), (1,)), ((), ())))
        h = 0.5 * h * (1.0 + jax.lax.erf(h * _INV_SQRT2))
        hw = h * wgt_ref[0, pl.ds(base, T_BLK), :]
        w2b = w2_ref[0]
        contrib = jax.lax.dot_general(hw, w2b, (((1,), (1,)), ((), ())))

        @pl.when(i == 0)
        def _():
            acc_ref[pl.ds(base, T_BLK), :] = contrib

        @pl.when(i > 0)
        def _():
            acc_ref[pl.ds(base, T_BLK), :] += contrib

        @pl.when(i == n_i - 1)
        def _():
            def scatter_grp(o, carry):
                rbase = base + o * 16
                for k in range(16):
                    t = idx_ref[e * T + rbase + k]
                    out_ref[pl.ds(t, 1), :] += acc_ref[pl.ds(rbase + k, 1), :]
                return carry
            jax.lax.fori_loop(0, ngrp, scatter_grp, 0)


def _router_call(x, sim_matrix, gates):
    T, C = x.shape
    E = sim_matrix.shape[1]
    T_BLK = 256
    n_tb = T // T_BLK
    return pl.pallas_call(
        _router_body,
        grid=(n_tb,),
        in_specs=[
            pl.BlockSpec((T_BLK, C), lambda tb: (tb, 0)),
            pl.BlockSpec((C, E), lambda tb: (0, 0)),
            pl.BlockSpec((1, E), lambda tb: (0, 0)),
        ],
        out_specs=pl.BlockSpec((E, T_BLK), lambda tb: (0, tb)),
        out_shape=jax.ShapeDtypeStruct((E, T), jnp.float32),
    )(x, sim_matrix, gates.reshape(1, E))


def _compact_call(wgt):
    E, T = wgt.shape
    active = wgt > 0.0
    counts = jnp.sum(active, axis=1).astype(jnp.int32)
    idx = jnp.argsort(~active, axis=1, stable=True).astype(jnp.int32)
    wgt_c = jnp.take_along_axis(wgt, idx, axis=1)
    counts16 = jnp.broadcast_to(counts[:, None], (E, _LANES))
    return idx, wgt_c, counts16


def _ffn_call(counts, idx, x, wgt_c, w1, w2):
    T, C = x.shape
    E, I, _ = w1.shape
    T_BLK = 512
    I_BLK = 1024
    n_tb = T // T_BLK
    n_i = I // I_BLK
    return pl.pallas_call(
        functools.partial(_ffn_body, T_BLK=T_BLK, n_i=n_i),
        grid_spec=pltpu.PrefetchScalarGridSpec(
            num_scalar_prefetch=2,
            grid=(E, n_i, n_tb),
            in_specs=[
                pl.BlockSpec((T, C), lambda e, i, tb, c_r, x_r: (0, 0)),
                pl.BlockSpec((1, T, 1), lambda e, i, tb, c_r, x_r: (e, 0, 0)),
                pl.BlockSpec((1, I_BLK, C), lambda e, i, tb, c_r, x_r: (e, i, 0)),
                pl.BlockSpec((1, C, I_BLK), lambda e, i, tb, c_r, x_r: (e, 0, i)),
            ],
            out_specs=pl.BlockSpec((T, C), lambda e, i, tb, c_r, x_r: (0, 0)),
            scratch_shapes=[
                pltpu.VMEM((T, C), jnp.float32),
                pltpu.VMEM((T, C), jnp.float32),
            ],
        ),
        out_shape=jax.ShapeDtypeStruct((T, C), jnp.float32),
        compiler_params=pltpu.CompilerParams(
            dimension_semantics=("arbitrary", "arbitrary", "arbitrary"),
        ),
    )(counts, idx.reshape(E * T), x, wgt_c.reshape(E, T, 1), w1, w2)


def kernel(hidden_states, sim_matrix, gates, w1, w2):
    B, T, C = hidden_states.shape
    x = hidden_states.reshape(T, C)
    wgt = _router_call(x, sim_matrix, gates)
    idx, wgt_c, counts = _compact_call(wgt)
    out = _ffn_call(counts, idx, x, wgt_c, w1, w2)
    return out.reshape(B, T, C)


# speedup vs baseline: 1.5100x; 1.0262x over previous
import functools
import math

import jax
import jax.numpy as jnp
from jax import lax
from jax.experimental import pallas as pl
from jax.experimental.pallas import tpu as pltpu
from jax.experimental.pallas import tpu_sc as plsc


_INV_SQRT2 = 1.0 / math.sqrt(2.0)
_LANES = 16


def _router_body(x_ref, sim_ref, gates_ref, w_ref):
    x = x_ref[...]
    s = sim_ref[...]
    xnorm = jnp.sqrt(jnp.sum(x * x, axis=1, keepdims=True))
    xn = x / jnp.maximum(xnorm, 1e-12)
    snorm = jnp.sqrt(jnp.sum(s * s, axis=0, keepdims=True))
    sn = s / jnp.maximum(snorm, 1e-12)
    logits = jnp.dot(xn, sn)
    thr = jax.nn.sigmoid(gates_ref[...])
    a = jnp.maximum(logits - thr, 0.0)
    active = a > 0.0
    amax = jnp.max(a, axis=1, keepdims=True)
    ex = jnp.where(active, jnp.exp(a - amax), 0.0)
    tot = jnp.sum(ex, axis=1, keepdims=True)
    w_ref[...] = (ex / jnp.where(tot > 0.0, tot, 1.0)).T


def _compact_body(wgt_hbm, idx_hbm, wgtc_hbm, counts_hbm,
                  col_v, idx_v, wgtc_v, cnt_v, *, E, T, NC):
    wid = lax.axis_index("s") * NC + lax.axis_index("c")

    @pl.when(wid < E)
    def _():
        pltpu.sync_copy(wgt_hbm.at[wid], col_v)
        one = jnp.ones((_LANES,), jnp.int32)
        zero = jnp.zeros((_LANES,), jnp.int32)

        def chunk(j, off_vec):
            base = j * _LANES
            idx_v[pl.ds(base, _LANES)] = zero
            wgtc_v[pl.ds(base, _LANES)] = jnp.zeros((_LANES,), jnp.float32)
            v = col_v[pl.ds(base, _LANES)]
            m = v > 0.0
            mi = jnp.where(m, one, zero)
            ids = lax.iota(jnp.int32, _LANES) + base
            pos = off_vec + plsc.cumsum(mi) - 1
            plsc.store_scatter(idx_v, [pos], ids, mask=m)
            plsc.store_scatter(wgtc_v, [pos], v, mask=m)
            return off_vec + plsc.all_reduce_population_count(m)

        cnt_v[...] = lax.fori_loop(0, T // _LANES, chunk, zero)
        pltpu.sync_copy(cnt_v, counts_hbm.at[wid])
        pltpu.sync_copy(idx_v, idx_hbm.at[wid])
        pltpu.sync_copy(wgtc_v, wgtc_hbm.at[wid])


def _ffn_body(counts_ref, idx_ref, x_ref, wgt_ref, w1_ref, w2_ref, out_ref,
              xg_ref, acc_ref, *, T_BLK, n_i):
    e = pl.program_id(0)
    i = pl.program_id(1)
    tb = pl.program_id(2)
    T = x_ref.shape[0]

    @pl.when((e == 0) & (i == 0) & (tb == 0))
    def _():
        out_ref[...] = jnp.zeros_like(out_ref)
        xg_ref[...] = jnp.zeros_like(xg_ref)

    cnt = counts_ref[e, 0]
    base = tb * T_BLK

    @pl.when(base < cnt)
    def _():
        nrows = jnp.minimum(cnt - base, T_BLK)
        ngrp = (nrows + 15) // 16

        @pl.when(i == 0)
        def _():
            def gather_grp(o, carry):
                rbase = base + o * 16
                for k in range(16):
                    t = idx_ref[e * T + rbase + k]
                    xg_ref[pl.ds(rbase + k, 1), :] = x_ref[pl.ds(t, 1), :]
                return carry
            jax.lax.fori_loop(0, ngrp, gather_grp, 0)

        xb = xg_ref[pl.ds(base, T_BLK), :]
        w1b = w1_ref[0]
        h = jax.lax.dot_general(xb, w1b, (((1,), (1,)), ((), ())))
        h = 0.5 * h * (1.0 + jax.lax.erf(h * _INV_SQRT2))
        hw = h * wgt_ref[0, pl.ds(base, T_BLK), :]
        w2b = w2_ref[0]
        contrib = jax.lax.dot_general(hw, w2b, (((1,), (1,)), ((), ())))

        @pl.when(i == 0)
        def _():
            acc_ref[pl.ds(base, T_BLK), :] = contrib

        @pl.when(i > 0)
        def _():
            acc_ref[pl.ds(base, T_BLK), :] += contrib

        @pl.when(i == n_i - 1)
        def _():
            def scatter_grp(o, carry):
                rbase = base + o * 16
                for k in range(16):
                    t = idx_ref[e * T + rbase + k]
                    out_ref[pl.ds(t, 1), :] += acc_ref[pl.ds(rbase + k, 1), :]
                return carry
            jax.lax.fori_loop(0, ngrp, scatter_grp, 0)


def _router_call(x, sim_matrix, gates):
    T, C = x.shape
    E = sim_matrix.shape[1]
    T_BLK = 256
    n_tb = T // T_BLK
    return pl.pallas_call(
        _router_body,
        grid=(n_tb,),
        in_specs=[
            pl.BlockSpec((T_BLK, C), lambda tb: (tb, 0)),
            pl.BlockSpec((C, E), lambda tb: (0, 0)),
            pl.BlockSpec((1, E), lambda tb: (0, 0)),
        ],
        out_specs=pl.BlockSpec((E, T_BLK), lambda tb: (0, tb)),
        out_shape=jax.ShapeDtypeStruct((E, T), jnp.float32),
    )(x, sim_matrix, gates.reshape(1, E))


def _compact_call(wgt):
    E, T = wgt.shape
    active = wgt > 0.0
    counts = jnp.sum(active, axis=1).astype(jnp.int32)
    idx = jnp.argsort(~active, axis=1, stable=True).astype(jnp.int32)
    wgt_c = jnp.take_along_axis(wgt, idx, axis=1)
    counts16 = jnp.broadcast_to(counts[:, None], (E, _LANES))
    return idx, wgt_c, counts16


def _ffn_call(counts, idx, x, wgt_c, w1, w2):
    T, C = x.shape
    E, I, _ = w1.shape
    T_BLK = 1024
    I_BLK = 1024
    n_tb = T // T_BLK
    n_i = I // I_BLK
    return pl.pallas_call(
        functools.partial(_ffn_body, T_BLK=T_BLK, n_i=n_i),
        grid_spec=pltpu.PrefetchScalarGridSpec(
            num_scalar_prefetch=2,
            grid=(E, n_i, n_tb),
            in_specs=[
                pl.BlockSpec((T, C), lambda e, i, tb, c_r, x_r: (0, 0)),
                pl.BlockSpec((1, T, 1), lambda e, i, tb, c_r, x_r: (e, 0, 0)),
                pl.BlockSpec((1, I_BLK, C), lambda e, i, tb, c_r, x_r: (e, i, 0)),
                pl.BlockSpec((1, C, I_BLK), lambda e, i, tb, c_r, x_r: (e, 0, i)),
            ],
            out_specs=pl.BlockSpec((T, C), lambda e, i, tb, c_r, x_r: (0, 0)),
            scratch_shapes=[
                pltpu.VMEM((T, C), jnp.float32),
                pltpu.VMEM((T, C), jnp.float32),
            ],
        ),
        out_shape=jax.ShapeDtypeStruct((T, C), jnp.float32),
        compiler_params=pltpu.CompilerParams(
            dimension_semantics=("arbitrary", "arbitrary", "arbitrary"),
        ),
    )(counts, idx.reshape(E * T), x, wgt_c.reshape(E, T, 1), w1, w2)


def kernel(hidden_states, sim_matrix, gates, w1, w2):
    B, T, C = hidden_states.shape
    x = hidden_states.reshape(T, C)
    wgt = _router_call(x, sim_matrix, gates)
    idx, wgt_c, counts = _compact_call(wgt)
    out = _ffn_call(counts, idx, x, wgt_c, w1, w2)
    return out.reshape(B, T, C)


# speedup vs baseline: 1.5106x; 1.0004x over previous
import functools
import math

import jax
import jax.numpy as jnp
from jax.experimental import pallas as pl
from jax.experimental.pallas import tpu as pltpu


_INV_SQRT2 = 1.0 / math.sqrt(2.0)
_LANES = 16


def _router_body(x_ref, sim_ref, gates_ref, w_ref):
    x = x_ref[...]
    s = sim_ref[...]
    xnorm = jnp.sqrt(jnp.sum(x * x, axis=1, keepdims=True))
    xn = x / jnp.maximum(xnorm, 1e-12)
    snorm = jnp.sqrt(jnp.sum(s * s, axis=0, keepdims=True))
    sn = s / jnp.maximum(snorm, 1e-12)
    logits = jnp.dot(xn, sn)
    thr = jax.nn.sigmoid(gates_ref[...])
    a = jnp.maximum(logits - thr, 0.0)
    active = a > 0.0
    amax = jnp.max(a, axis=1, keepdims=True)
    ex = jnp.where(active, jnp.exp(a - amax), 0.0)
    tot = jnp.sum(ex, axis=1, keepdims=True)
    w_ref[...] = (ex / jnp.where(tot > 0.0, tot, 1.0)).T


def _ffn_body(counts_ref, idx_ref, x_ref, wgt_ref, w1_ref, w2_ref, out_ref,
              xg_ref, acc_ref, *, T_BLK, n_i):
    e = pl.program_id(0)
    i = pl.program_id(1)
    tb = pl.program_id(2)
    T = x_ref.shape[0]

    @pl.when((e == 0) & (i == 0) & (tb == 0))
    def _():
        out_ref[...] = jnp.zeros_like(out_ref)
        xg_ref[...] = jnp.zeros_like(xg_ref)

    cnt = counts_ref[e, 0]
    base = tb * T_BLK

    @pl.when(base < cnt)
    def _():
        nrows = jnp.minimum(cnt - base, T_BLK)
        ngrp = (nrows + 15) // 16

        @pl.when(i == 0)
        def _():
            def gather_grp(o, carry):
                rbase = base + o * 16
                for k in range(16):
                    t = idx_ref[e * T + rbase + k]
                    xg_ref[pl.ds(rbase + k, 1), :] = x_ref[pl.ds(t, 1), :]
                return carry
            jax.lax.fori_loop(0, ngrp, gather_grp, 0)

        xb = xg_ref[pl.ds(base, T_BLK), :]
        w1b = w1_ref[0]
        h = jax.lax.dot_general(xb, w1b, (((1,), (1,)), ((), ())))
        h = 0.5 * h * (1.0 + jax.lax.erf(h * _INV_SQRT2))
        hw = h * wgt_ref[0, pl.ds(base, T_BLK), :]
        w2b = w2_ref[0]
        contrib = jax.lax.dot_general(hw, w2b, (((1,), (1,)), ((), ())))

        @pl.when(i == 0)
        def _():
            acc_ref[pl.ds(base, T_BLK), :] = contrib

        @pl.when(i > 0)
        def _():
            acc_ref[pl.ds(base, T_BLK), :] += contrib

        @pl.when(i == n_i - 1)
        def _():
            def scatter_grp(o, carry):
                rbase = base + o * 16
                for k in range(16):
                    t = idx_ref[e * T + rbase + k]
                    out_ref[pl.ds(t, 1), :] += acc_ref[pl.ds(rbase + k, 1), :]
                return carry
            jax.lax.fori_loop(0, ngrp, scatter_grp, 0)


def _router_call(x, sim_matrix, gates):
    T, C = x.shape
    E = sim_matrix.shape[1]
    T_BLK = 256
    n_tb = T // T_BLK
    return pl.pallas_call(
        _router_body,
        grid=(n_tb,),
        in_specs=[
            pl.BlockSpec((T_BLK, C), lambda tb: (tb, 0)),
            pl.BlockSpec((C, E), lambda tb: (0, 0)),
            pl.BlockSpec((1, E), lambda tb: (0, 0)),
        ],
        out_specs=pl.BlockSpec((E, T_BLK), lambda tb: (0, tb)),
        out_shape=jax.ShapeDtypeStruct((E, T), jnp.float32),
    )(x, sim_matrix, gates.reshape(1, E))


def _compact_call(wgt):
    E, T = wgt.shape
    active = wgt > 0.0
    counts = jnp.sum(active, axis=1).astype(jnp.int32)
    idx = jnp.argsort(~active, axis=1, stable=True).astype(jnp.int32)
    wgt_c = jnp.take_along_axis(wgt, idx, axis=1)
    counts16 = jnp.broadcast_to(counts[:, None], (E, _LANES))
    return idx, wgt_c, counts16


def _ffn_call(counts, idx, x, wgt_c, w1, w2):
    T, C = x.shape
    E, I, _ = w1.shape
    T_BLK = 1024
    I_BLK = 1024
    n_tb = T // T_BLK
    n_i = I // I_BLK
    return pl.pallas_call(
        functools.partial(_ffn_body, T_BLK=T_BLK, n_i=n_i),
        grid_spec=pltpu.PrefetchScalarGridSpec(
            num_scalar_prefetch=2,
            grid=(E, n_i, n_tb),
            in_specs=[
                pl.BlockSpec((T, C), lambda e, i, tb, c_r, x_r: (0, 0)),
                pl.BlockSpec((1, T, 1), lambda e, i, tb, c_r, x_r: (e, 0, 0)),
                pl.BlockSpec((1, I_BLK, C), lambda e, i, tb, c_r, x_r: (e, i, 0)),
                pl.BlockSpec((1, C, I_BLK), lambda e, i, tb, c_r, x_r: (e, 0, i)),
            ],
            out_specs=pl.BlockSpec((T, C), lambda e, i, tb, c_r, x_r: (0, 0)),
            scratch_shapes=[
                pltpu.VMEM((T, C), jnp.float32),
                pltpu.VMEM((T, C), jnp.float32),
            ],
        ),
        out_shape=jax.ShapeDtypeStruct((T, C), jnp.float32),
        compiler_params=pltpu.CompilerParams(
            dimension_semantics=("arbitrary", "arbitrary", "arbitrary"),
        ),
    )(counts, idx.reshape(E * T), x, wgt_c.reshape(E, T, 1), w1, w2)


def kernel(hidden_states, sim_matrix, gates, w1, w2):
    B, T, C = hidden_states.shape
    x = hidden_states.reshape(T, C)
    wgt = _router_call(x, sim_matrix, gates)
    idx, wgt_c, counts = _compact_call(wgt)
    out = _ffn_call(counts, idx, x, wgt_c, w1, w2)
    return out.reshape(B, T, C)
